# baseline (speedup 1.0000x reference)
import functools

import jax
import jax.numpy as jnp
from jax import lax
from jax.experimental import pallas as pl
from jax.experimental.pallas import tpu as pltpu
from jax.experimental.pallas import tpu_sc as plsc

_VOCAB = 100000
_DIM = 64
_B = 4096
_L_TXT = 200
_L_TOP = 20

_NC = 2
_NS = 16
_NW = _NC * _NS
_BT = _B // _NW

_LC = 2
_TXT_CHUNKS = _L_TXT // _LC
_TOP_CHUNKS = _L_TOP // _LC
_DT = _DIM // 8


def _transpose_chunk(rows_v, tbuf, p):
  iota = lax.iota(jnp.int32, 16)
  p_vec = jnp.full((16,), p, jnp.int32)
  d_vecs = [iota + 16 * j for j in range(4)]

  for i in range(_LC):
    i_vec = jnp.full((16,), i, jnp.int32)

    def body(b, carry):
      b_vec = jnp.zeros((16,), jnp.int32) + b
      for j in range(4):
        v = rows_v[p, i, b, pl.ds(16 * j, 16)]
        plsc.store_scatter(tbuf, [p_vec, i_vec, d_vecs[j], b_vec], v)
      return carry

    lax.fori_loop(0, _BT, body, 0)


def _run_stripe(idx_hbm, out_hbm, table_hbm, idx_v, rows_v, tbuf, sems,
                bt, n):
  b0 = bt * _BT

  def fire_idx(g, p):
    pltpu.async_copy(idx_hbm.at[pl.ds(g * _LC, _LC), pl.ds(b0, _BT)],
                     idx_v.at[p], sems[p][0])

  def fire_gathers(g, p):
    del g
    pltpu.make_async_copy(idx_hbm.at[pl.ds(0, _LC), pl.ds(b0, _BT)],
                          idx_v.at[p], sems[p][0]).wait()
    for i in range(_LC):
      pltpu.async_copy(table_hbm.at[idx_v.at[p, i]], rows_v.at[p, i],
                       sems[p][1])

  def process(g, p, drain_writes):
    for i in range(_LC):
      pltpu.make_async_copy(table_hbm.at[pl.ds(0, _BT)], rows_v.at[p, i],
                            sems[p][1]).wait()
    if drain_writes is not None:
      @pl.when(drain_writes)
      def _():
        _wait_writes(g, p)
    _transpose_chunk(rows_v, tbuf, p)
    for dt in range(_DT):
      pltpu.async_copy(tbuf.at[p, :, pl.ds(dt * 8, 8)],
                       out_hbm.at[pl.ds(g * _LC, _LC), dt, bt], sems[p][2])

  def _wait_writes(g, p):
    for dt in range(_DT):
      pltpu.make_async_copy(tbuf.at[p, :, pl.ds(dt * 8, 8)],
                            out_hbm.at[pl.ds(g * _LC, _LC), dt, bt],
                            sems[p][2]).wait()

  fire_idx(0, 0)
  fire_idx(1, 1)
  fire_gathers(0, 0)
  fire_gathers(1, 1)
  process(0, 0, None)
  if n >= 3:
    fire_idx(2, 0)

  n_iter = n - 2
  n_pairs = n_iter // 2

  def iter_block(g, p):
    q = 1 - p
    fire_gathers(g + 1, q)
    process(g, p, g >= 2)
    @pl.when(g + 2 <= n - 1)
    def _():
      fire_idx(g + 2, p)

  if n_pairs > 0:
    def pair_body(k, carry):
      g0 = 1 + 2 * k
      for b in range(2):
        iter_block(g0 + b, (1 + b) % 2)
      return carry

    lax.fori_loop(0, n_pairs, pair_body, 0)

  for g in range(1 + 2 * n_pairs, n - 1):
    iter_block(g, g % 2)

  p_last = (n - 1) % 2
  process(n - 1, p_last, n - 1 >= 2)
  _wait_writes(n - 2, 1 - p_last)
  _wait_writes(n - 1, p_last)


_mesh = plsc.VectorSubcoreMesh(core_axis_name="c", subcore_axis_name="s")


@functools.partial(
    pl.kernel,
    mesh=_mesh,
    compiler_params=pltpu.CompilerParams(use_tc_tiling_on_sc=False,
                                         needs_layout_passes=False),
    out_type=(
        jax.ShapeDtypeStruct((_L_TXT, 8, _NW, 8, _BT), jnp.float32),
        jax.ShapeDtypeStruct((_L_TOP, 8, _NW, 8, _BT), jnp.float32),
    ),
    scratch_types=[
        pltpu.VMEM((2, _LC, _BT), jnp.int32),
        pltpu.VMEM((2, _LC, _BT, _DIM), jnp.float32),
        pltpu.VMEM((2, _LC, _DIM, _BT), jnp.float32),
        pltpu.SemaphoreType.DMA,
        pltpu.SemaphoreType.DMA,
        pltpu.SemaphoreType.DMA,
        pltpu.SemaphoreType.DMA,
        pltpu.SemaphoreType.DMA,
        pltpu.SemaphoreType.DMA,
    ],
)
def _embed_lookup(text_hbm, topic_hbm, table_hbm, txt_out, top_out,
                  idx_v, rows_v, tbuf, si0, sg0, sw0, si1, sg1, sw1):
  bt = lax.axis_index("s") * _NC + lax.axis_index("c")
  sems = ((si0, sg0, sw0), (si1, sg1, sw1))
  _run_stripe(text_hbm, txt_out, table_hbm, idx_v, rows_v, tbuf, sems,
              bt, _TXT_CHUNKS)
  _run_stripe(topic_hbm, top_out, table_hbm, idx_v, rows_v, tbuf, sems,
              bt, _TOP_CHUNKS)


def kernel(text, topic, table):
  t5, p5 = _embed_lookup(text.T.astype(jnp.int32), topic.T.astype(jnp.int32),
                         table)
  txt = jnp.transpose(t5, (2, 4, 0, 1, 3)).reshape(_B, _L_TXT, _DIM)
  top = jnp.transpose(p5, (2, 4, 0, 1, 3)).reshape(_B, _L_TOP, _DIM)
  return (txt, top)


# speedup vs baseline: 5.0780x; 5.0780x over previous
import functools

import jax
import jax.numpy as jnp
from jax import lax
from jax.experimental import pallas as pl
from jax.experimental.pallas import tpu as pltpu
from jax.experimental.pallas import tpu_sc as plsc

_VOCAB = 100000
_DIM = 64
_B = 4096
_L_TXT = 200
_L_TOP = 20

_NC = 2
_NS = 16
_NW = _NC * _NS
_BT = _B // _NW

_LC = 2
_TXT_CHUNKS = _L_TXT // _LC
_TOP_CHUNKS = _L_TOP // _LC
_DT = _DIM // 8


def _transpose_chunk(rows_v, tbuf, p):
  iota = lax.iota(jnp.int32, 16)
  p_vec = jnp.full((16,), p, jnp.int32)
  base = [iota * _BT + 16 * _BT * j for j in range(4)]

  for i in range(_LC):
    i_vec = jnp.full((16,), i, jnp.int32)

    @functools.partial(plsc.parallel_loop, 0, _BT, unroll=4,
                       carry=tuple(base))
    def _(b, flat):
      for j in range(4):
        v = rows_v[p, i, b, pl.ds(16 * j, 16)]
        plsc.store_scatter(tbuf, [p_vec, i_vec, flat[j]], v)
      return tuple(f + 1 for f in flat)


def _run_stripe(idx_hbm, out_hbm, table_hbm, idx_v, rows_v, tbuf, sems,
                bt, n):
  b0 = bt * _BT

  def fire_idx(g, p):
    pltpu.async_copy(idx_hbm.at[pl.ds(g * _LC, _LC), pl.ds(b0, _BT)],
                     idx_v.at[p], sems[p][0])

  def fire_gathers(g, p):
    del g
    pltpu.make_async_copy(idx_hbm.at[pl.ds(0, _LC), pl.ds(b0, _BT)],
                          idx_v.at[p], sems[p][0]).wait()
    for i in range(_LC):
      pltpu.async_copy(table_hbm.at[idx_v.at[p, i]], rows_v.at[p, i],
                       sems[p][1])

  def process(g, p, drain_writes):
    for i in range(_LC):
      pltpu.make_async_copy(table_hbm.at[pl.ds(0, _BT)], rows_v.at[p, i],
                            sems[p][1]).wait()
    if drain_writes is not None:
      @pl.when(drain_writes)
      def _():
        _wait_writes(g, p)
    _transpose_chunk(rows_v, tbuf, p)
    for dt in range(_DT):
      pltpu.async_copy(tbuf.at[p, :, pl.ds(dt * 8 * _BT, 8 * _BT)],
                       out_hbm.at[pl.ds(g * _LC, _LC), dt, bt], sems[p][2])

  def _wait_writes(g, p):
    for dt in range(_DT):
      pltpu.make_async_copy(tbuf.at[p, :, pl.ds(dt * 8 * _BT, 8 * _BT)],
                            out_hbm.at[pl.ds(g * _LC, _LC), dt, bt],
                            sems[p][2]).wait()

  fire_idx(0, 0)
  fire_idx(1, 1)
  fire_gathers(0, 0)
  fire_gathers(1, 1)
  process(0, 0, None)
  if n >= 3:
    fire_idx(2, 0)

  n_iter = n - 2
  n_pairs = n_iter // 2

  def iter_block(g, p):
    q = 1 - p
    fire_gathers(g + 1, q)
    process(g, p, g >= 2)
    @pl.when(g + 2 <= n - 1)
    def _():
      fire_idx(g + 2, p)

  if n_pairs > 0:
    def pair_body(k, carry):
      g0 = 1 + 2 * k
      for b in range(2):
        iter_block(g0 + b, (1 + b) % 2)
      return carry

    lax.fori_loop(0, n_pairs, pair_body, 0)

  for g in range(1 + 2 * n_pairs, n - 1):
    iter_block(g, g % 2)

  p_last = (n - 1) % 2
  process(n - 1, p_last, n - 1 >= 2)
  _wait_writes(n - 2, 1 - p_last)
  _wait_writes(n - 1, p_last)


_mesh = plsc.VectorSubcoreMesh(core_axis_name="c", subcore_axis_name="s")


@functools.partial(
    pl.kernel,
    mesh=_mesh,
    compiler_params=pltpu.CompilerParams(use_tc_tiling_on_sc=False,
                                         needs_layout_passes=False),
    out_type=(
        jax.ShapeDtypeStruct((_L_TXT, 8, _NW, 8 * _BT), jnp.float32),
        jax.ShapeDtypeStruct((_L_TOP, 8, _NW, 8 * _BT), jnp.float32),
    ),
    scratch_types=[
        pltpu.VMEM((2, _LC, _BT), jnp.int32),
        pltpu.VMEM((2, _LC, _BT, _DIM), jnp.float32),
        pltpu.VMEM((2, _LC, _DIM * _BT), jnp.float32),
        pltpu.SemaphoreType.DMA,
        pltpu.SemaphoreType.DMA,
        pltpu.SemaphoreType.DMA,
        pltpu.SemaphoreType.DMA,
        pltpu.SemaphoreType.DMA,
        pltpu.SemaphoreType.DMA,
    ],
)
def _embed_lookup(text_hbm, topic_hbm, table_hbm, txt_out, top_out,
                  idx_v, rows_v, tbuf, si0, sg0, sw0, si1, sg1, sw1):
  bt = lax.axis_index("s") * _NC + lax.axis_index("c")
  sems = ((si0, sg0, sw0), (si1, sg1, sw1))
  _run_stripe(text_hbm, txt_out, table_hbm, idx_v, rows_v, tbuf, sems,
              bt, _TXT_CHUNKS)
  _run_stripe(topic_hbm, top_out, table_hbm, idx_v, rows_v, tbuf, sems,
              bt, _TOP_CHUNKS)


def kernel(text, topic, table):
  t5, p5 = _embed_lookup(text.T.astype(jnp.int32), topic.T.astype(jnp.int32),
                         table)
  t5 = t5.reshape(_L_TXT, 8, _NW, 8, _BT)
  p5 = p5.reshape(_L_TOP, 8, _NW, 8, _BT)
  txt = jnp.transpose(t5, (2, 4, 0, 1, 3)).reshape(_B, _L_TXT, _DIM)
  top = jnp.transpose(p5, (2, 4, 0, 1, 3)).reshape(_B, _L_TOP, _DIM)
  return (txt, top)
